# baseline (speedup 1.0000x reference)
import functools

import jax
import jax.numpy as jnp
from jax import lax
from jax.experimental import pallas as pl
from jax.experimental.pallas import tpu as pltpu
from jax.experimental.pallas import tpu_sc as plsc

N = 10000
E = 320000
D = 128

NC = 2
NS = 16
NW = NC * NS
C = 64
K0 = 139
K1 = 175
EPW = C * 175
EPAD = NS * C * (K0 + K1)
ZROWS = 640
ZPAD = NS * ZROWS

_HI = -65536


def _proj_body(h_ref, w2_ref, bias_ref, d_ref, ab16_ref, hp_ref):
  ab = (
      lax.dot_general(
          w2_ref[...], h_ref[...], (((1,), (1,)), ((), ())),
          preferred_element_type=jnp.float32,
      )
      + bias_ref[...]
  )
  abu = lax.bitcast_convert_type(ab, jnp.int32)
  ab16_ref[...] = (abu[0:1] & _HI) | lax.shift_right_logical(abu[1:2], 16)
  hp_ref[...] = h_ref[...] * d_ref[...]


def _add_body(p_ref, d_ref, out_ref):
  out_ref[...] = (p_ref[0] + p_ref[1]) * d_ref[...]


def _extract_ids(packed_v, cb, sidx_v, didx_v):
  for g in range(C // 16):
    v = packed_v[pl.ds(cb + g * 16, 16)]
    sidx_v[pl.ds(g * 16, 16)] = v & 0xFFFF
    didx_v[pl.ds(g * 16, 16)] = lax.shift_right_logical(v, 16)


def _compute_chunk(ab16_v, w_v, sidx_v, didx_v, rows_v):
  for g in range(C // 16):
    si = sidx_v[pl.ds(g * 16, 16)]
    di = didx_v[pl.ds(g * 16, 16)]
    pd = plsc.load_gather(ab16_v, [di])
    ps = plsc.load_gather(ab16_v, [si])
    a = plsc.bitcast(pd & _HI, jnp.float32)
    b = plsc.bitcast(lax.shift_left(ps, 16), jnp.float32)
    x = a + b
    t = 1.0 - 2.0 / (jnp.exp(jnp.abs(x) * 2.0) + 1.0)
    w_v[pl.ds(g * 16, 16)] = jnp.where(x < 0.0, -t, t)

  def scale(e, _):
    e2 = e * 2
    ws0 = plsc.load_gather(w_v, [jnp.full((16,), e2, jnp.int32)])
    ws1 = plsc.load_gather(w_v, [jnp.full((16,), e2 + 1, jnp.int32)])
    for j in range(8):
      rows_v[e2, pl.ds(j * 16, 16)] = rows_v[e2, pl.ds(j * 16, 16)] * ws0
    for j in range(8):
      rows_v[e2 + 1, pl.ds(j * 16, 16)] = (
          rows_v[e2 + 1, pl.ds(j * 16, 16)] * ws1)
    return 0

  lax.fori_loop(0, C // 2, scale, 0)


def _edge_body(packed_hbm, ab16_hbm, hp_hbm, out_hbm,
               packed_v, ab16_v, rows0, rows1, rows2, w_v,
               sidx0, sidx1, sidx2, didx0, didx1, didx2,
               semg0, semg1, semg2, sems0, sems1, sems2, z_sh):
  cid = lax.axis_index("c")
  sid = lax.axis_index("s")
  nchunk = jnp.where(cid == 0, K0, K1)
  base = pl.multiple_of(
      cid * (NS * C * K0) + sid * (C * nchunk), 8)

  pltpu.sync_copy(ab16_hbm, ab16_v)
  pltpu.sync_copy(packed_hbm.at[pl.ds(base, EPW)], packed_v)

  zeros16 = jnp.zeros((16,), jnp.float32)

  def zero_row(i, _):
    for j in range(8):
      rows0[i, pl.ds(j * 16, 16)] = zeros16
    return 0

  lax.fori_loop(0, C, zero_row, 0)
  for k in range(ZROWS // C):
    pltpu.sync_copy(rows0, z_sh.at[pl.ds(sid * ZROWS + k * C, C)])
  plsc.subcore_barrier()

  bufs = ((rows0, sidx0, didx0, semg0, sems0),
          (rows1, sidx1, didx1, semg1, sems1),
          (rows2, sidx2, didx2, semg2, sems2))

  _extract_ids(packed_v, 0, sidx0, didx0)
  pltpu.async_copy(hp_hbm.at[sidx0], rows0, semg0)
  _extract_ids(packed_v, C, sidx1, didx1)
  pltpu.async_copy(hp_hbm.at[sidx1], rows1, semg1)

  def step(i, cur, nx2):
    rows_c, sidx_c, didx_c, semg_c, sems_c = cur
    rows_n, sidx_n, didx_n, semg_n, sems_n = nx2

    @pl.when(i > 0)
    def _():
      pltpu.make_async_copy(rows_n, z_sh.at[didx_n], sems_n).wait()

    @pl.when(i + 2 < nchunk)
    def _():
      cb = pl.multiple_of((i + 2) * C, 8)
      _extract_ids(packed_v, cb, sidx_n, didx_n)
      pltpu.async_copy(hp_hbm.at[sidx_n], rows_n, semg_n)

    pltpu.make_async_copy(hp_hbm.at[sidx_c], rows_c, semg_c).wait()
    _compute_chunk(ab16_v, w_v, sidx_c, didx_c, rows_c)
    pltpu.async_copy(rows_c, z_sh.at[didx_c], sems_c, add=True)

  def chunk(i, _):
    r = lax.rem(i, 3)

    @pl.when(r == 0)
    def _():
      step(i, bufs[0], bufs[2])

    @pl.when(r == 1)
    def _():
      step(i, bufs[1], bufs[0])

    @pl.when(r == 2)
    def _():
      step(i, bufs[2], bufs[1])

    return 0

  lax.fori_loop(0, nchunk, chunk, 0)
  pltpu.make_async_copy(rows0, z_sh.at[didx0], sems0).wait()
  plsc.subcore_barrier()

  pltpu.sync_copy(
      z_sh.at[pl.ds(sid * ZROWS, ZROWS)],
      out_hbm.at[cid, pl.ds(sid * ZROWS, ZROWS)],
  )


_edge_call = functools.partial(
    pl.kernel,
    out_type=jax.ShapeDtypeStruct((NC, ZPAD, D), jnp.float32),
    mesh=plsc.VectorSubcoreMesh(
        core_axis_name="c", subcore_axis_name="s", num_cores=NC,
        num_subcores=NS,
    ),
    scratch_types=[
        pltpu.VMEM((EPW,), jnp.int32),
        pltpu.VMEM((N,), jnp.int32),
        pltpu.VMEM((C, D), jnp.float32),
        pltpu.VMEM((C, D), jnp.float32),
        pltpu.VMEM((C, D), jnp.float32),
        pltpu.VMEM((C,), jnp.float32),
        pltpu.VMEM((C,), jnp.int32),
        pltpu.VMEM((C,), jnp.int32),
        pltpu.VMEM((C,), jnp.int32),
        pltpu.VMEM((C,), jnp.int32),
        pltpu.VMEM((C,), jnp.int32),
        pltpu.VMEM((C,), jnp.int32),
        pltpu.SemaphoreType.DMA,
        pltpu.SemaphoreType.DMA,
        pltpu.SemaphoreType.DMA,
        pltpu.SemaphoreType.DMA,
        pltpu.SemaphoreType.DMA,
        pltpu.SemaphoreType.DMA,
        pltpu.VMEM_SHARED((ZPAD, D), jnp.float32),
    ],
    compiler_params=pltpu.CompilerParams(needs_layout_passes=False),
)(_edge_body)


@jax.jit
def kernel(h, edge_index, d, W_gate, b_gate):
  w2 = W_gate.reshape(2, D)
  bias = jnp.concatenate([b_gate, jnp.zeros((1,), jnp.float32)]).reshape(2, 1)
  d2 = d.reshape(N, 1)

  ab16, hp = pl.pallas_call(
      _proj_body,
      out_shape=(
          jax.ShapeDtypeStruct((1, N), jnp.int32),
          jax.ShapeDtypeStruct((N, D), jnp.float32),
      ),
  )(h, w2, bias, d2)

  packed = edge_index[0] | (edge_index[1] << 16)
  packed = jnp.concatenate(
      [packed, jnp.full((EPAD - E,), N << 16, jnp.int32)])

  partials = _edge_call(packed, ab16.reshape(N), hp)

  z = pl.pallas_call(
      _add_body,
      grid=(10,),
      in_specs=[
          pl.BlockSpec((2, N // 10, D), lambda i: (0, i, 0)),
          pl.BlockSpec((N // 10, 1), lambda i: (i, 0)),
      ],
      out_specs=pl.BlockSpec((N // 10, D), lambda i: (i, 0)),
      out_shape=jax.ShapeDtypeStruct((N, D), jnp.float32),
  )(partials, d2)
  return z


# speedup vs baseline: 1.1044x; 1.1044x over previous
import functools

import jax
import jax.numpy as jnp
from jax import lax
from jax.experimental import pallas as pl
from jax.experimental.pallas import tpu as pltpu
from jax.experimental.pallas import tpu_sc as plsc

N = 10000
E = 320000
D = 128

NC = 2
NS = 16
NW = NC * NS
C = 64
K0 = 175
K1 = 139
EPW = C * 175
EPAD = NS * C * (K0 + K1)
ZROWS = 640
ZPAD = NS * ZROWS

_HI = -65536


def _proj_body(h_ref, w2_ref, bias_ref, d_ref, ab16_ref, hp_ref):
  ab = (
      lax.dot_general(
          w2_ref[...], h_ref[...], (((1,), (1,)), ((), ())),
          preferred_element_type=jnp.float32,
      )
      + bias_ref[...]
  )
  abu = lax.bitcast_convert_type(ab, jnp.int32)
  ab16_ref[...] = (abu[0:1] & _HI) | lax.shift_right_logical(abu[1:2], 16)
  hp_ref[...] = h_ref[...] * d_ref[...]


def _add_body(p_ref, d_ref, out_ref):
  out_ref[...] = (p_ref[0] + p_ref[1]) * d_ref[...]


def _extract_ids(packed_v, cb, sidx_v, didx_v):
  for g in range(C // 16):
    v = packed_v[pl.ds(cb + g * 16, 16)]
    sidx_v[pl.ds(g * 16, 16)] = v & 0xFFFF
    didx_v[pl.ds(g * 16, 16)] = lax.shift_right_logical(v, 16)


def _compute_chunk(ab16_v, w_v, sidx_v, didx_v, rows_v):
  for g in range(C // 16):
    si = sidx_v[pl.ds(g * 16, 16)]
    di = didx_v[pl.ds(g * 16, 16)]
    pd = plsc.load_gather(ab16_v, [di])
    ps = plsc.load_gather(ab16_v, [si])
    a = plsc.bitcast(pd & _HI, jnp.float32)
    b = plsc.bitcast(lax.shift_left(ps, 16), jnp.float32)
    x = a + b
    t = 1.0 - 2.0 / (jnp.exp(jnp.abs(x) * 2.0) + 1.0)
    w_v[pl.ds(g * 16, 16)] = jnp.where(x < 0.0, -t, t)

  def scale(e, _):
    e2 = e * 2
    ws0 = plsc.load_gather(w_v, [jnp.full((16,), e2, jnp.int32)])
    ws1 = plsc.load_gather(w_v, [jnp.full((16,), e2 + 1, jnp.int32)])
    for j in range(8):
      rows_v[e2, pl.ds(j * 16, 16)] = rows_v[e2, pl.ds(j * 16, 16)] * ws0
    for j in range(8):
      rows_v[e2 + 1, pl.ds(j * 16, 16)] = (
          rows_v[e2 + 1, pl.ds(j * 16, 16)] * ws1)
    return 0

  lax.fori_loop(0, C // 2, scale, 0)


def _edge_body(packed_hbm, ab16_hbm, hp_hbm, out_hbm,
               packed_v, ab16_v, rows0, rows1, rows2, w_v,
               sidx0, sidx1, sidx2, didx0, didx1, didx2,
               semg0, semg1, semg2, sems0, sems1, sems2, z_sh):
  cid = lax.axis_index("c")
  sid = lax.axis_index("s")
  nchunk = jnp.where(cid == 0, K0, K1)
  base = pl.multiple_of(
      cid * (NS * C * K0) + sid * (C * nchunk), 8)

  pltpu.sync_copy(ab16_hbm, ab16_v)
  pltpu.sync_copy(packed_hbm.at[pl.ds(base, EPW)], packed_v)

  zeros16 = jnp.zeros((16,), jnp.float32)

  def zero_row(i, _):
    for j in range(8):
      rows0[i, pl.ds(j * 16, 16)] = zeros16
    return 0

  lax.fori_loop(0, C, zero_row, 0)
  for k in range(ZROWS // C):
    pltpu.sync_copy(rows0, z_sh.at[pl.ds(sid * ZROWS + k * C, C)])
  plsc.subcore_barrier()

  bufs = ((rows0, sidx0, didx0, semg0, sems0),
          (rows1, sidx1, didx1, semg1, sems1),
          (rows2, sidx2, didx2, semg2, sems2))

  _extract_ids(packed_v, 0, sidx0, didx0)
  pltpu.async_copy(hp_hbm.at[sidx0], rows0, semg0)
  _extract_ids(packed_v, C, sidx1, didx1)
  pltpu.async_copy(hp_hbm.at[sidx1], rows1, semg1)

  def step(i, cur, nx2):
    rows_c, sidx_c, didx_c, semg_c, sems_c = cur
    rows_n, sidx_n, didx_n, semg_n, sems_n = nx2

    @pl.when(i > 0)
    def _():
      pltpu.make_async_copy(rows_n, z_sh.at[didx_n], sems_n).wait()

    @pl.when(i + 2 < nchunk)
    def _():
      cb = pl.multiple_of((i + 2) * C, 8)
      _extract_ids(packed_v, cb, sidx_n, didx_n)
      pltpu.async_copy(hp_hbm.at[sidx_n], rows_n, semg_n)

    pltpu.make_async_copy(hp_hbm.at[sidx_c], rows_c, semg_c).wait()
    _compute_chunk(ab16_v, w_v, sidx_c, didx_c, rows_c)
    pltpu.async_copy(rows_c, z_sh.at[didx_c], sems_c, add=True)

  def chunk(i, _):
    r = lax.rem(i, 3)

    @pl.when(r == 0)
    def _():
      step(i, bufs[0], bufs[2])

    @pl.when(r == 1)
    def _():
      step(i, bufs[1], bufs[0])

    @pl.when(r == 2)
    def _():
      step(i, bufs[2], bufs[1])

    return 0

  lax.fori_loop(0, nchunk, chunk, 0)
  pltpu.make_async_copy(rows0, z_sh.at[didx0], sems0).wait()
  plsc.subcore_barrier()

  pltpu.sync_copy(
      z_sh.at[pl.ds(sid * ZROWS, ZROWS)],
      out_hbm.at[cid, pl.ds(sid * ZROWS, ZROWS)],
  )


_edge_call = functools.partial(
    pl.kernel,
    out_type=jax.ShapeDtypeStruct((NC, ZPAD, D), jnp.float32),
    mesh=plsc.VectorSubcoreMesh(
        core_axis_name="c", subcore_axis_name="s", num_cores=NC,
        num_subcores=NS,
    ),
    scratch_types=[
        pltpu.VMEM((EPW,), jnp.int32),
        pltpu.VMEM((N,), jnp.int32),
        pltpu.VMEM((C, D), jnp.float32),
        pltpu.VMEM((C, D), jnp.float32),
        pltpu.VMEM((C, D), jnp.float32),
        pltpu.VMEM((C,), jnp.float32),
        pltpu.VMEM((C,), jnp.int32),
        pltpu.VMEM((C,), jnp.int32),
        pltpu.VMEM((C,), jnp.int32),
        pltpu.VMEM((C,), jnp.int32),
        pltpu.VMEM((C,), jnp.int32),
        pltpu.VMEM((C,), jnp.int32),
        pltpu.SemaphoreType.DMA,
        pltpu.SemaphoreType.DMA,
        pltpu.SemaphoreType.DMA,
        pltpu.SemaphoreType.DMA,
        pltpu.SemaphoreType.DMA,
        pltpu.SemaphoreType.DMA,
        pltpu.VMEM_SHARED((ZPAD, D), jnp.float32),
    ],
    compiler_params=pltpu.CompilerParams(needs_layout_passes=False),
)(_edge_body)


@jax.jit
def kernel(h, edge_index, d, W_gate, b_gate):
  w2 = W_gate.reshape(2, D)
  bias = jnp.concatenate([b_gate, jnp.zeros((1,), jnp.float32)]).reshape(2, 1)
  d2 = d.reshape(N, 1)

  ab16, hp = pl.pallas_call(
      _proj_body,
      out_shape=(
          jax.ShapeDtypeStruct((1, N), jnp.int32),
          jax.ShapeDtypeStruct((N, D), jnp.float32),
      ),
  )(h, w2, bias, d2)

  packed = edge_index[0] | (edge_index[1] << 16)
  packed = jnp.concatenate(
      [packed, jnp.full((EPAD - E,), N << 16, jnp.int32)])

  partials = _edge_call(packed, ab16.reshape(N), hp)

  z = pl.pallas_call(
      _add_body,
      grid=(10,),
      in_specs=[
          pl.BlockSpec((2, N // 10, D), lambda i: (0, i, 0)),
          pl.BlockSpec((N // 10, 1), lambda i: (i, 0)),
      ],
      out_specs=pl.BlockSpec((N // 10, D), lambda i: (i, 0)),
      out_shape=jax.ShapeDtypeStruct((N, D), jnp.float32),
  )(partials, d2)
  return z
